# Initial kernel scaffold; baseline (speedup 1.0000x reference)
import jax
import jax.numpy as jnp
from jax.experimental import pallas as pl


def kernel(features, A, W0, b0, W1, b1, W2, b2, Wf0, bf0, Wf1, bf1):
    raise NotImplementedError("write your pallas kernel here")


# speedup vs baseline: 2.1479x; 2.1479x over previous
import jax
import jax.numpy as jnp
from jax.experimental import pallas as pl

NSTACK = 3
NNODE = 25
TDIM = 300
TPAD = 304
TGRP = 8
RCHUNK = TGRP * NNODE
TTILE = 152
RB = TTILE * NNODE
NCH = RB // RCHUNK
HID = 256


def _gin_body(f_ref, bd_ref, mt_ref, w0_ref, b0_ref, w1_ref, b1_ref,
              w2_ref, b2_ref, out_ref):
    f = f_ref[0]
    bd = bd_ref[...]

    def bd_apply(x):
        return jnp.concatenate(
            [jnp.dot(bd, x[k * RCHUNK:(k + 1) * RCHUNK, :],
                     preferred_element_type=jnp.float32)
             for k in range(NCH)], axis=0)

    agg0 = bd_apply(f)
    acc = None
    for s in range(NSTACK):
        h = jnp.maximum(
            jnp.dot(agg0, w0_ref[s], preferred_element_type=jnp.float32)
            + b0_ref[s:s + 1, :], 0.0)
        h = jnp.maximum(
            jnp.dot(bd_apply(h), w1_ref[s], preferred_element_type=jnp.float32)
            + b1_ref[s:s + 1, :], 0.0)
        g = jnp.dot(h, w2_ref[s], preferred_element_type=jnp.float32)
        g2 = g.reshape(TTILE, NNODE)
        o = jnp.maximum(
            jnp.dot(g2, mt_ref[...], preferred_element_type=jnp.float32)
            + b2_ref[s:s + 1, :], 0.0)
        acc = o if acc is None else acc + o
    out_ref[0] = acc * (1.0 / NSTACK)


def _mlp_body(x_ref, wf0_ref, bf0_ref, wf1_ref, bf1_ref, out_ref):
    hfc = jnp.maximum(
        jnp.dot(x_ref[...], wf0_ref[...], preferred_element_type=jnp.float32)
        + bf0_ref[...], 0.0)
    out_ref[...] = (jnp.dot(hfc, wf1_ref[...],
                            preferred_element_type=jnp.float32)
                    + bf1_ref[...])


def kernel(features, A, W0, b0, W1, b1, W2, b2, Wf0, bf0, Wf1, bf1):
    B = features.shape[0]
    m_hat = A + jnp.eye(NNODE, dtype=A.dtype)
    bd = jnp.kron(jnp.eye(TGRP, dtype=A.dtype), m_hat)
    mt = m_hat.T

    f2 = features.reshape(B, TDIM * NNODE, 3)
    fpad = jnp.pad(f2, ((0, 0), (0, (TPAD - TDIM) * NNODE), (0, 0)))

    gin = pl.pallas_call(
        _gin_body,
        grid=(B, TPAD // TTILE),
        in_specs=[
            pl.BlockSpec((1, RB, 3), lambda b, j: (b, j, 0)),
            pl.BlockSpec((RCHUNK, RCHUNK), lambda b, j: (0, 0)),
            pl.BlockSpec((NNODE, NNODE), lambda b, j: (0, 0)),
            pl.BlockSpec((NSTACK, 3, HID), lambda b, j: (0, 0, 0)),
            pl.BlockSpec((NSTACK, HID), lambda b, j: (0, 0)),
            pl.BlockSpec((NSTACK, HID, HID), lambda b, j: (0, 0, 0)),
            pl.BlockSpec((NSTACK, HID), lambda b, j: (0, 0)),
            pl.BlockSpec((NSTACK, HID, 1), lambda b, j: (0, 0, 0)),
            pl.BlockSpec((NSTACK, 1), lambda b, j: (0, 0)),
        ],
        out_specs=pl.BlockSpec((1, TTILE, NNODE), lambda b, j: (b, j, 0)),
        out_shape=jax.ShapeDtypeStruct((B, TPAD, NNODE), jnp.float32),
    )(fpad, bd, mt, W0, b0, W1, b1, W2, b2)

    pooled = gin[:, :TDIM, :].reshape(B, TDIM * NNODE)
    logits = pl.pallas_call(
        _mlp_body,
        out_shape=jax.ShapeDtypeStruct((B, 60), jnp.float32),
    )(pooled, Wf0, bf0.reshape(1, -1), Wf1, bf1.reshape(1, -1))
    return logits


# speedup vs baseline: 4.1787x; 1.9455x over previous
import jax
import jax.numpy as jnp
from jax.experimental import pallas as pl

NSTACK = 3
NNODE = 25
TDIM = 300
TPAD = 304
TGRP = 8
RCHUNK = TGRP * NNODE
TTILE = 152
RB = TTILE * NNODE
NCH = RB // RCHUNK
HID = 256


def _gin_body(f_ref, bd_ref, mt_ref, w0_ref, b0_ref, w1_ref, b1_ref,
              w2_ref, b2_ref, out_ref):
    j = pl.program_id(1)
    rows = jax.lax.broadcasted_iota(jnp.int32, (RB, 1), 0)
    limit = jnp.where(j == TPAD // TTILE - 1, RB - (TPAD - TDIM) * NNODE, RB)
    f = jnp.where(rows < limit, f_ref[0], 0.0)
    bd = bd_ref[...]

    def bd_apply(x):
        return jnp.concatenate(
            [jnp.dot(bd, x[k * RCHUNK:(k + 1) * RCHUNK, :],
                     preferred_element_type=jnp.float32)
             for k in range(NCH)], axis=0)

    agg0 = bd_apply(f)
    acc = None
    for s in range(NSTACK):
        h = jnp.maximum(
            jnp.dot(agg0, w0_ref[s], preferred_element_type=jnp.float32)
            + b0_ref[s:s + 1, :], 0.0)
        h = jnp.maximum(
            jnp.dot(bd_apply(h), w1_ref[s], preferred_element_type=jnp.float32)
            + b1_ref[s:s + 1, :], 0.0)
        g = jnp.dot(h, w2_ref[s], preferred_element_type=jnp.float32)
        g2 = g.reshape(TTILE, NNODE)
        o = jnp.maximum(
            jnp.dot(g2, mt_ref[...], preferred_element_type=jnp.float32)
            + b2_ref[s:s + 1, :], 0.0)
        acc = o if acc is None else acc + o
    out_ref[0] = acc * (1.0 / NSTACK)


def _mlp_body(x_ref, wf0_ref, bf0_ref, wf1_ref, bf1_ref, out_ref):
    hfc = jnp.maximum(
        jnp.dot(x_ref[...], wf0_ref[...], preferred_element_type=jnp.float32)
        + bf0_ref[...], 0.0)
    out_ref[...] = (jnp.dot(hfc, wf1_ref[...],
                            preferred_element_type=jnp.float32)
                    + bf1_ref[...])


def kernel(features, A, W0, b0, W1, b1, W2, b2, Wf0, bf0, Wf1, bf1):
    B = features.shape[0]
    m_hat = A + jnp.eye(NNODE, dtype=A.dtype)
    bd = jnp.kron(jnp.eye(TGRP, dtype=A.dtype), m_hat)
    mt = m_hat.T

    f2 = features.reshape(B, TDIM * NNODE, 3)

    gin = pl.pallas_call(
        _gin_body,
        grid=(B, TPAD // TTILE),
        in_specs=[
            pl.BlockSpec((1, RB, 3), lambda b, j: (b, j, 0)),
            pl.BlockSpec((RCHUNK, RCHUNK), lambda b, j: (0, 0)),
            pl.BlockSpec((NNODE, NNODE), lambda b, j: (0, 0)),
            pl.BlockSpec((NSTACK, 3, HID), lambda b, j: (0, 0, 0)),
            pl.BlockSpec((NSTACK, HID), lambda b, j: (0, 0)),
            pl.BlockSpec((NSTACK, HID, HID), lambda b, j: (0, 0, 0)),
            pl.BlockSpec((NSTACK, HID), lambda b, j: (0, 0)),
            pl.BlockSpec((NSTACK, HID, 1), lambda b, j: (0, 0, 0)),
            pl.BlockSpec((NSTACK, 1), lambda b, j: (0, 0)),
        ],
        out_specs=pl.BlockSpec((1, TTILE, NNODE), lambda b, j: (b, j, 0)),
        out_shape=jax.ShapeDtypeStruct((B, TDIM, NNODE), jnp.float32),
    )(f2, bd, mt, W0, b0, W1, b1, W2, b2)

    pooled = gin.reshape(B, TDIM * NNODE)
    logits = pl.pallas_call(
        _mlp_body,
        out_shape=jax.ShapeDtypeStruct((B, 60), jnp.float32),
    )(pooled, Wf0, bf0.reshape(1, -1), Wf1, bf1.reshape(1, -1))
    return logits


# speedup vs baseline: 4.6449x; 1.1116x over previous
import jax
import jax.numpy as jnp
from jax.experimental import pallas as pl

NSTACK = 3
NNODE = 25
TDIM = 300
TPAD = 304
TGRP = 8
RCHUNK = TGRP * NNODE
TTILE = 152
RB = TTILE * NNODE
NCH = RB // RCHUNK
HID = 256


def _gin_body(f_ref, bd_ref, w0_ref, b0_ref, w1_ref, b1_ref,
              w2_ref, b2_ref, out_ref):
    j = pl.program_id(1)
    rows = jax.lax.broadcasted_iota(jnp.int32, (RB, 1), 0)
    limit = jnp.where(j == TPAD // TTILE - 1, RB - (TPAD - TDIM) * NNODE, RB)
    f = jnp.where(rows < limit, f_ref[0], 0.0)
    bd = bd_ref[...]

    def bd_apply(x):
        return jnp.concatenate(
            [jnp.dot(bd, x[k * RCHUNK:(k + 1) * RCHUNK, :],
                     preferred_element_type=jnp.float32)
             for k in range(NCH)], axis=0)

    agg0 = bd_apply(f)
    acc = None
    for s in range(NSTACK):
        h = jnp.maximum(
            jnp.dot(agg0, w0_ref[s], preferred_element_type=jnp.float32)
            + b0_ref[s:s + 1, :], 0.0)
        h = jnp.maximum(
            jnp.dot(bd_apply(h), w1_ref[s], preferred_element_type=jnp.float32)
            + b1_ref[s:s + 1, :], 0.0)
        g = jnp.dot(h, w2_ref[s], preferred_element_type=jnp.float32)
        o = jnp.maximum(bd_apply(g) + b2_ref[s:s + 1, :], 0.0)
        acc = o if acc is None else acc + o
    out_ref[0] = acc * (1.0 / NSTACK)


def _mlp_body(x_ref, wf0_ref, bf0_ref, wf1_ref, bf1_ref, out_ref):
    hfc = jnp.maximum(
        jnp.dot(x_ref[...], wf0_ref[...], preferred_element_type=jnp.float32)
        + bf0_ref[...], 0.0)
    out_ref[...] = (jnp.dot(hfc, wf1_ref[...],
                            preferred_element_type=jnp.float32)
                    + bf1_ref[...])


def kernel(features, A, W0, b0, W1, b1, W2, b2, Wf0, bf0, Wf1, bf1):
    B = features.shape[0]
    m_hat = A + jnp.eye(NNODE, dtype=A.dtype)
    bd = jnp.kron(jnp.eye(TGRP, dtype=A.dtype), m_hat)

    f2 = features.reshape(B, TDIM * NNODE, 3)

    gin = pl.pallas_call(
        _gin_body,
        grid=(B, TPAD // TTILE),
        in_specs=[
            pl.BlockSpec((1, RB, 3), lambda b, j: (b, j, 0)),
            pl.BlockSpec((RCHUNK, RCHUNK), lambda b, j: (0, 0)),
            pl.BlockSpec((NSTACK, 3, HID), lambda b, j: (0, 0, 0)),
            pl.BlockSpec((NSTACK, HID), lambda b, j: (0, 0)),
            pl.BlockSpec((NSTACK, HID, HID), lambda b, j: (0, 0, 0)),
            pl.BlockSpec((NSTACK, HID), lambda b, j: (0, 0)),
            pl.BlockSpec((NSTACK, HID, 1), lambda b, j: (0, 0, 0)),
            pl.BlockSpec((NSTACK, 1), lambda b, j: (0, 0)),
        ],
        out_specs=pl.BlockSpec((1, RB, 1), lambda b, j: (b, j, 0)),
        out_shape=jax.ShapeDtypeStruct((B, TDIM * NNODE, 1), jnp.float32),
    )(f2, bd, W0, b0, W1, b1, W2, b2)

    pooled = gin.reshape(B, TDIM * NNODE)
    logits = pl.pallas_call(
        _mlp_body,
        out_shape=jax.ShapeDtypeStruct((B, 60), jnp.float32),
    )(pooled, Wf0, bf0.reshape(1, -1), Wf1, bf1.reshape(1, -1))
    return logits
